# baseline (speedup 1.0000x reference)
import jax
import jax.numpy as jnp
from jax.experimental import pallas as pl
from jax.experimental.pallas import tpu as pltpu

IN_FEATURES = 1024
NUM_PATTERNS = 100000
BLOCK_ROWS = 1000


def _body(q_ref, mem_ref, out_ref, best_ref):
    i = pl.program_id(0)

    @pl.when(i == 0)
    def _():
        best_ref[0] = -jnp.inf

    q = q_ref[...]
    block = mem_ref[...]
    dots = jnp.sum(block * q, axis=1, keepdims=True)
    sumsq = jnp.sum(block * block, axis=1, keepdims=True)
    q_norm = jnp.sqrt(jnp.sum(q * q))
    denom = jnp.maximum(q_norm * jnp.sqrt(sumsq), 1e-8)
    sims = dots / denom
    local_max = jnp.max(sims)

    @pl.when(local_max > best_ref[0])
    def _():
        best_ref[0] = local_max
        r = sims.shape[0]
        iota = jax.lax.broadcasted_iota(jnp.int32, (r, 1), 0)
        idx = jnp.min(jnp.where(sims == local_max, iota, r))
        out_ref[...] = mem_ref[pl.ds(idx, 1), :]


def kernel(query, memory):
    k, d = memory.shape
    q2 = query.reshape(1, d)
    grid = k // BLOCK_ROWS
    out = pl.pallas_call(
        _body,
        grid=(grid,),
        in_specs=[
            pl.BlockSpec((1, d), lambda i: (0, 0)),
            pl.BlockSpec((BLOCK_ROWS, d), lambda i: (i, 0)),
        ],
        out_specs=pl.BlockSpec((1, d), lambda i: (0, 0)),
        out_shape=jax.ShapeDtypeStruct((1, d), jnp.float32),
        scratch_shapes=[pltpu.SMEM((1,), jnp.float32)],
    )(q2, memory)
    return out.reshape(d)


# speedup vs baseline: 1.2127x; 1.2127x over previous
import jax
import jax.numpy as jnp
from jax.experimental import pallas as pl
from jax.experimental.pallas import tpu as pltpu

IN_FEATURES = 1024
NUM_PATTERNS = 100000
BLOCK_ROWS = 2000


def _body(q_ref, mem_ref, out_ref, best_ref):
    i = pl.program_id(0)

    @pl.when(i == 0)
    def _():
        best_ref[0] = -jnp.inf

    q = q_ref[...]
    block = mem_ref[...]
    dn = (((1,), (1,)), ((), ()))
    dots = jax.lax.dot_general(block, q, dn,
                               preferred_element_type=jnp.float32)
    ones = jnp.ones(q.shape, jnp.float32)
    sumsq = jax.lax.dot_general(block * block, ones, dn,
                                preferred_element_type=jnp.float32)
    q_norm = jnp.sqrt(jnp.sum(q * q))
    denom = jnp.maximum(q_norm * jnp.sqrt(sumsq), 1e-8)
    sims = dots / denom
    local_max = jnp.max(sims)

    @pl.when(local_max > best_ref[0])
    def _():
        best_ref[0] = local_max
        r = sims.shape[0]
        iota = jax.lax.broadcasted_iota(jnp.int32, (r, 1), 0)
        idx = jnp.min(jnp.where(sims == local_max, iota, r))
        out_ref[...] = mem_ref[pl.ds(idx, 1), :]


def kernel(query, memory):
    k, d = memory.shape
    q2 = query.reshape(1, d)
    grid = k // BLOCK_ROWS
    out = pl.pallas_call(
        _body,
        grid=(grid,),
        in_specs=[
            pl.BlockSpec((1, d), lambda i: (0, 0)),
            pl.BlockSpec((BLOCK_ROWS, d), lambda i: (i, 0)),
        ],
        out_specs=pl.BlockSpec((1, d), lambda i: (0, 0)),
        out_shape=jax.ShapeDtypeStruct((1, d), jnp.float32),
        scratch_shapes=[pltpu.SMEM((1,), jnp.float32)],
    )(q2, memory)
    return out.reshape(d)
